# baseline (speedup 1.0000x reference)
import functools

import jax
import jax.numpy as jnp
from jax import lax
from jax.experimental import pallas as pl
from jax.experimental.pallas import tpu as pltpu
from jax.experimental.pallas import tpu_sc as plsc

D = 32
MIN_RATING = 0.5
MAX_RATING = 5.0

NC = 2
NS = 16
NW = NC * NS
CHUNK = 128


def _gather_body(uids_hbm, iids_hbm, uemb_hbm, iemb_hbm, ubias_hbm, ibias_hbm,
                 ue_out, ie_out, ub_out, ib_out,
                 uidx, iidx, urows, irows, ubv, ibv, sem):
    n_chunks = uidx.shape[0]
    b_per_w = n_chunks * CHUNK
    wid = lax.axis_index("s") * NC + lax.axis_index("c")
    base = wid * b_per_w

    pltpu.sync_copy(uids_hbm.at[pl.ds(wid * n_chunks, n_chunks)], uidx)
    pltpu.sync_copy(iids_hbm.at[pl.ds(wid * n_chunks, n_chunks)], iidx)

    copies = []
    for j in range(n_chunks):
        sl = pl.ds(j * CHUNK, CHUNK)
        copies.append(pltpu.async_copy(uemb_hbm.at[uidx.at[j]], urows.at[sl], sem))
        copies.append(pltpu.async_copy(iemb_hbm.at[iidx.at[j]], irows.at[sl], sem))
        copies.append(pltpu.async_copy(ubias_hbm.at[uidx.at[j]], ubv.at[sl], sem))
        copies.append(pltpu.async_copy(ibias_hbm.at[iidx.at[j]], ibv.at[sl], sem))
    for c in copies:
        c.wait()

    out_sl = pl.ds(base, b_per_w)
    pltpu.sync_copy(urows, ue_out.at[out_sl])
    pltpu.sync_copy(irows, ie_out.at[out_sl])
    pltpu.sync_copy(ubv, ub_out.at[out_sl])
    pltpu.sync_copy(ibv, ib_out.at[out_sl])


@functools.partial(jax.jit, static_argnames=("batch",))
def _sc_gather(uids2, iids2, uemb, iemb, ubias1, ibias1, *, batch):
    b_per_w = batch // NW
    n_chunks = b_per_w // CHUNK
    mesh = plsc.VectorSubcoreMesh(core_axis_name="c", subcore_axis_name="s")
    f = pl.kernel(
        _gather_body,
        out_type=[
            jax.ShapeDtypeStruct((batch, D), jnp.float32),
            jax.ShapeDtypeStruct((batch, D), jnp.float32),
            jax.ShapeDtypeStruct((batch,), jnp.float32),
            jax.ShapeDtypeStruct((batch,), jnp.float32),
        ],
        mesh=mesh,
        compiler_params=pltpu.CompilerParams(use_tc_tiling_on_sc=False),
        scratch_types=[
            pltpu.VMEM((n_chunks, CHUNK), jnp.int32),
            pltpu.VMEM((n_chunks, CHUNK), jnp.int32),
            pltpu.VMEM((b_per_w, D), jnp.float32),
            pltpu.VMEM((b_per_w, D), jnp.float32),
            pltpu.VMEM((b_per_w,), jnp.float32),
            pltpu.VMEM((b_per_w,), jnp.float32),
            pltpu.SemaphoreType.DMA,
        ],
    )
    return f(uids2, iids2, uemb, iemb, ubias1, ibias1)


def _mlp_body(ue_ref, ie_ref, ub_ref, ib_ref, w1a_ref, w1b_ref, b1_ref,
              w2_ref, b2_ref, w3_ref, b3_ref, out_ref):
    h = ue_ref[...] @ w1a_ref[...] + ie_ref[...] @ w1b_ref[...] + b1_ref[...]
    h = jnp.maximum(h, 0.0)
    h = jnp.maximum(h @ w2_ref[...] + b2_ref[...], 0.0)
    p = h @ w3_ref[...] + b3_ref[...] + ub_ref[...] + ib_ref[...]
    out_ref[...] = jnp.clip(p, MIN_RATING, MAX_RATING)


@functools.partial(jax.jit, static_argnames=("batch",))
def _tc_mlp(ue, ie, ub, ib, w1a, w1b, b1, w2, b2, w3, b3, *, batch):
    blk = 2048
    grid = (batch // blk,)
    full = lambda shape: pl.BlockSpec(shape, lambda i: (0, 0))
    return pl.pallas_call(
        _mlp_body,
        grid=grid,
        in_specs=[
            pl.BlockSpec((blk, D), lambda i: (i, 0)),
            pl.BlockSpec((blk, D), lambda i: (i, 0)),
            pl.BlockSpec((blk, 1), lambda i: (i, 0)),
            pl.BlockSpec((blk, 1), lambda i: (i, 0)),
            full((D, 32)),
            full((D, 32)),
            full((1, 32)),
            full((32, 16)),
            full((1, 16)),
            full((16, 1)),
            full((1, 1)),
        ],
        out_specs=pl.BlockSpec((blk, 1), lambda i: (i, 0)),
        out_shape=jax.ShapeDtypeStruct((batch, 1), jnp.float32),
    )(ue, ie, ub, ib, w1a, w1b, b1, w2, b2, w3, b3)


def kernel(user_ids, item_ids, user_emb, item_emb, user_bias, item_bias,
           W1, b1, W2, b2, W3, b3):
    batch = user_ids.shape[0]
    uids2 = user_ids.astype(jnp.int32).reshape(batch // CHUNK, CHUNK)
    iids2 = item_ids.astype(jnp.int32).reshape(batch // CHUNK, CHUNK)
    ue, ie, ub, ib = _sc_gather(
        uids2, iids2, user_emb, item_emb,
        user_bias.reshape(-1), item_bias.reshape(-1), batch=batch)
    return _tc_mlp(
        ue, ie, ub.reshape(batch, 1), ib.reshape(batch, 1),
        W1[:D], W1[D:], b1.reshape(1, -1), W2, b2.reshape(1, -1),
        W3, b3.reshape(1, 1), batch=batch)


# speedup vs baseline: 1.0003x; 1.0003x over previous
import functools

import jax
import jax.numpy as jnp
from jax import lax
from jax.experimental import pallas as pl
from jax.experimental.pallas import tpu as pltpu
from jax.experimental.pallas import tpu_sc as plsc

D = 32
MIN_RATING = 0.5
MAX_RATING = 5.0

NC = 2
NS = 16
NW = NC * NS
CHUNK = 128


def _gather_body(uids_hbm, iids_hbm, uemb_hbm, iemb_hbm, ubias_hbm, ibias_hbm,
                 ue_out, ie_out, ub_out, ib_out,
                 uidx, iidx, urows, irows, ubv, ibv, sem):
    n_chunks = uidx.shape[0]
    b_per_w = n_chunks * CHUNK
    wid = lax.axis_index("s") * NC + lax.axis_index("c")
    base = wid * b_per_w

    pltpu.sync_copy(uids_hbm.at[pl.ds(wid * n_chunks, n_chunks)], uidx)
    pltpu.sync_copy(iids_hbm.at[pl.ds(wid * n_chunks, n_chunks)], iidx)

    copies = []
    for j in range(n_chunks):
        sl = pl.ds(j * CHUNK, CHUNK)
        copies.append(pltpu.async_copy(uemb_hbm.at[uidx.at[j]], urows.at[sl], sem))
        copies.append(pltpu.async_copy(iemb_hbm.at[iidx.at[j]], irows.at[sl], sem))
        copies.append(pltpu.async_copy(ubias_hbm.at[uidx.at[j]], ubv.at[sl], sem))
        copies.append(pltpu.async_copy(ibias_hbm.at[iidx.at[j]], ibv.at[sl], sem))
    for c in copies:
        c.wait()

    out_sl = pl.ds(base, b_per_w)
    pltpu.sync_copy(urows, ue_out.at[out_sl])
    pltpu.sync_copy(irows, ie_out.at[out_sl])
    pltpu.sync_copy(ubv, ub_out.at[out_sl])
    pltpu.sync_copy(ibv, ib_out.at[out_sl])


@functools.partial(jax.jit, static_argnames=("batch",))
def _sc_gather(uids2, iids2, uemb, iemb, ubias, ibias, *, batch):
    b_per_w = batch // NW
    n_chunks = b_per_w // CHUNK
    mesh = plsc.VectorSubcoreMesh(core_axis_name="c", subcore_axis_name="s")
    f = pl.kernel(
        _gather_body,
        out_type=[
            jax.ShapeDtypeStruct((batch, D), jnp.float32),
            jax.ShapeDtypeStruct((batch, D), jnp.float32),
            jax.ShapeDtypeStruct((batch,), jnp.float32),
            jax.ShapeDtypeStruct((batch,), jnp.float32),
        ],
        mesh=mesh,
        compiler_params=pltpu.CompilerParams(use_tc_tiling_on_sc=False),
        scratch_types=[
            pltpu.VMEM((n_chunks, CHUNK), jnp.int32),
            pltpu.VMEM((n_chunks, CHUNK), jnp.int32),
            pltpu.VMEM((b_per_w, D), jnp.float32),
            pltpu.VMEM((b_per_w, D), jnp.float32),
            pltpu.VMEM((b_per_w,), jnp.float32),
            pltpu.VMEM((b_per_w,), jnp.float32),
            pltpu.SemaphoreType.DMA,
        ],
    )
    return f(uids2, iids2, uemb, iemb, ubias, ibias)


def _mlp_body(ue_ref, ie_ref, ub_ref, ib_ref, w1a_ref, w1b_ref, b1_ref,
              w2_ref, b2_ref, w3_ref, b3_ref, out_ref):
    h = ue_ref[...] @ w1a_ref[...] + ie_ref[...] @ w1b_ref[...] + b1_ref[...]
    h = jnp.maximum(h, 0.0)
    h = jnp.maximum(h @ w2_ref[...] + b2_ref[...], 0.0)
    p = h @ w3_ref[...] + b3_ref[...] + ub_ref[...] + ib_ref[...]
    out_ref[...] = jnp.clip(p, MIN_RATING, MAX_RATING)


@functools.partial(jax.jit, static_argnames=("batch",))
def _tc_mlp(ue, ie, ub, ib, w1a, w1b, b1, w2, b2, w3, b3, *, batch):
    blk = 2048
    grid = (batch // blk,)
    full = lambda shape: pl.BlockSpec(shape, lambda i: (0, 0))
    return pl.pallas_call(
        _mlp_body,
        grid=grid,
        in_specs=[
            pl.BlockSpec((blk, D), lambda i: (i, 0)),
            pl.BlockSpec((blk, D), lambda i: (i, 0)),
            pl.BlockSpec((blk, 1), lambda i: (i, 0)),
            pl.BlockSpec((blk, 1), lambda i: (i, 0)),
            full((D, 32)),
            full((D, 32)),
            full((1, 32)),
            full((32, 16)),
            full((1, 16)),
            full((16, 1)),
            full((1, 1)),
        ],
        out_specs=pl.BlockSpec((blk, 1), lambda i: (i, 0)),
        out_shape=jax.ShapeDtypeStruct((batch, 1), jnp.float32),
    )(ue, ie, ub, ib, w1a, w1b, b1, w2, b2, w3, b3)


def kernel(user_ids, item_ids, user_emb, item_emb, user_bias, item_bias,
           W1, b1, W2, b2, W3, b3):
    batch = user_ids.shape[0]
    uids2 = user_ids.astype(jnp.int32).reshape(batch // CHUNK, CHUNK)
    iids2 = item_ids.astype(jnp.int32).reshape(batch // CHUNK, CHUNK)
    ue, ie, ub, ib = _sc_gather(
        uids2, iids2, user_emb, item_emb,
        user_bias.T.reshape(-1), item_bias.T.reshape(-1), batch=batch)
    return _tc_mlp(
        ue, ie, ub.reshape(batch, 1), ib.reshape(batch, 1),
        W1[:D], W1[D:], b1.reshape(1, -1), W2, b2.reshape(1, -1),
        W3, b3.reshape(1, 1), batch=batch)
